# Initial kernel scaffold; baseline (speedup 1.0000x reference)
import jax
import jax.numpy as jnp
from jax.experimental import pallas as pl


def kernel(x, edge_attr, Wq1, bq1, Wk1, bk1, Wv1, bv1, We1, Ws1, bs1, Wb1, Wq2, bq2, Wk2, bk2, Wv2, bv2, We2, Ws2, bs2, Wb2, Wq3, bq3, Wk3, bk3, Wv3, bv3, We3, Ws3, bs3, Wb3, ln1_g, ln1_b, ln2_g, ln2_b, ln3_g, ln3_b, gate_W, gate_b, fc_W, fc_b, edge_index, batch):
    raise NotImplementedError("write your pallas kernel here")


# speedup vs baseline: 3.2780x; 3.2780x over previous
import math
import jax
import jax.numpy as jnp
from jax.experimental import pallas as pl

N = 10000
E = 160000
H = 8
ED = 4
G = 8


def _mm_bias(x, W, b, BN):
    n, K = x.shape
    M = W.shape[1]
    b2 = b.reshape(1, M)

    def kern(x_ref, w_ref, b_ref, o_ref):
        o_ref[...] = jnp.dot(x_ref[...], w_ref[...],
                             preferred_element_type=jnp.float32) + b_ref[...]

    return pl.pallas_call(
        kern,
        grid=(n // BN,),
        in_specs=[
            pl.BlockSpec((BN, K), lambda i: (i, 0)),
            pl.BlockSpec((K, M), lambda i: (0, 0)),
            pl.BlockSpec((1, M), lambda i: (0, 0)),
        ],
        out_specs=pl.BlockSpec((BN, M), lambda i: (i, 0)),
        out_shape=jax.ShapeDtypeStruct((n, M), jnp.float32),
    )(x, W, b2)


def _alpha_edges(qd, ks, ea, We, ch, BE):
    HC = H * ch
    scale = 1.0 / math.sqrt(ch)

    def kern(qd_ref, ks_ref, ea_ref, we_ref, o_ref):
        ep = jnp.dot(ea_ref[...], we_ref[...],
                     preferred_element_type=jnp.float32)
        t = qd_ref[...] * (ks_ref[...] + ep)
        r0 = jax.lax.broadcasted_iota(jnp.int32, (HC, H), 0) // ch
        r1 = jax.lax.broadcasted_iota(jnp.int32, (HC, H), 1)
        seg = (r0 == r1).astype(jnp.float32)
        o_ref[...] = jnp.dot(t, seg, preferred_element_type=jnp.float32) * scale

    return pl.pallas_call(
        kern,
        grid=(E // BE,),
        in_specs=[
            pl.BlockSpec((BE, HC), lambda i: (i, 0)),
            pl.BlockSpec((BE, HC), lambda i: (i, 0)),
            pl.BlockSpec((BE, ED), lambda i: (i, 0)),
            pl.BlockSpec((ED, HC), lambda i: (0, 0)),
        ],
        out_specs=pl.BlockSpec((BE, H), lambda i: (i, 0)),
        out_shape=jax.ShapeDtypeStruct((E, H), jnp.float32),
    )(qd, ks, ea, We)


def _weighted_edges(vs, a, ea, We, ch, BE):
    HC = H * ch

    def kern(vs_ref, a_ref, ea_ref, we_ref, o_ref):
        ep = jnp.dot(ea_ref[...], we_ref[...],
                     preferred_element_type=jnp.float32)
        r0 = jax.lax.broadcasted_iota(jnp.int32, (H, HC), 0)
        r1 = jax.lax.broadcasted_iota(jnp.int32, (H, HC), 1) // ch
        expand = (r0 == r1).astype(jnp.float32)
        w = jnp.dot(a_ref[...], expand,
                    preferred_element_type=jnp.float32)
        t = w * (vs_ref[...] + ep)
        c0 = jax.lax.broadcasted_iota(jnp.int32, (HC, ch), 0) % ch
        c1 = jax.lax.broadcasted_iota(jnp.int32, (HC, ch), 1)
        fold = (c0 == c1).astype(jnp.float32) * (1.0 / H)
        o_ref[...] = jnp.dot(t, fold, preferred_element_type=jnp.float32)

    return pl.pallas_call(
        kern,
        grid=(E // BE,),
        in_specs=[
            pl.BlockSpec((BE, HC), lambda i: (i, 0)),
            pl.BlockSpec((BE, H), lambda i: (i, 0)),
            pl.BlockSpec((BE, ED), lambda i: (i, 0)),
            pl.BlockSpec((ED, HC), lambda i: (0, 0)),
        ],
        out_specs=pl.BlockSpec((BE, ch), lambda i: (i, 0)),
        out_shape=jax.ShapeDtypeStruct((E, ch), jnp.float32),
    )(vs, a, ea, We)


def _combine(outm, xr, Wb, ln_g, ln_b, gate_W, gate_b, ch, BN):
    n = outm.shape[0]
    W1 = Wb[0:ch]
    W2 = Wb[ch:2 * ch]
    W3 = Wb[2 * ch:3 * ch]

    def kern(o_ref, x_ref, w1_ref, w2_ref, w3_ref, g_ref, bb_ref,
             gw_ref, gb_ref, h_ref, gate_ref):
        o = o_ref[...]
        xr_ = x_ref[...]
        logits = (jnp.dot(o, w1_ref[...], preferred_element_type=jnp.float32)
                  + jnp.dot(xr_, w2_ref[...], preferred_element_type=jnp.float32)
                  + jnp.dot(o - xr_, w3_ref[...], preferred_element_type=jnp.float32))
        bgate = jax.nn.sigmoid(logits)
        h = bgate * xr_ + (1.0 - bgate) * o
        e = jnp.where(h > 0, h, jnp.exp(h) - 1.0)
        mu = jnp.mean(e, axis=-1, keepdims=True)
        var = jnp.mean((e - mu) ** 2, axis=-1, keepdims=True)
        ln = (e - mu) / jnp.sqrt(var + 1e-5) * g_ref[...] + bb_ref[...]
        h_ref[...] = ln
        gate_ref[...] = jnp.dot(ln, gw_ref[...],
                                preferred_element_type=jnp.float32) + gb_ref[...]

    return pl.pallas_call(
        kern,
        grid=(n // BN,),
        in_specs=[
            pl.BlockSpec((BN, ch), lambda i: (i, 0)),
            pl.BlockSpec((BN, ch), lambda i: (i, 0)),
            pl.BlockSpec((ch, 1), lambda i: (0, 0)),
            pl.BlockSpec((ch, 1), lambda i: (0, 0)),
            pl.BlockSpec((ch, 1), lambda i: (0, 0)),
            pl.BlockSpec((1, ch), lambda i: (0, 0)),
            pl.BlockSpec((1, ch), lambda i: (0, 0)),
            pl.BlockSpec((ch, 1), lambda i: (0, 0)),
            pl.BlockSpec((1, 1), lambda i: (0, 0)),
        ],
        out_specs=[
            pl.BlockSpec((BN, ch), lambda i: (i, 0)),
            pl.BlockSpec((BN, 1), lambda i: (i, 0)),
        ],
        out_shape=[
            jax.ShapeDtypeStruct((n, ch), jnp.float32),
            jax.ShapeDtypeStruct((n, 1), jnp.float32),
        ],
    )(outm, xr, W1, W2, W3, ln_g.reshape(1, ch), ln_b.reshape(1, ch),
      gate_W, gate_b.reshape(1, 1))


def _seg_softmax(logits, seg, num_segments):
    m = jax.ops.segment_max(logits, seg, num_segments=num_segments)
    m = jnp.where(jnp.isfinite(m), m, 0.0)
    ex = jnp.exp(logits - m[seg])
    s = jax.ops.segment_sum(ex, seg, num_segments=num_segments)
    return ex / (s[seg] + 1e-16)


def _layer(x, src, dst, ea, Wq, bq, Wk, bk, Wv, bv, We, Ws, bs, Wb,
           ln_g, ln_b, gate_W, gate_b, ch, BE):
    q = _mm_bias(x, Wq, bq, 400)
    k = _mm_bias(x, Wk, bk, 400)
    v = _mm_bias(x, Wv, bv, 400)
    xr = _mm_bias(x, Ws, bs, 1000)

    qd = jnp.take(q, dst, axis=0)
    ks = jnp.take(k, src, axis=0)
    vs = jnp.take(v, src, axis=0)

    alpha = _alpha_edges(qd, ks, ea, We, ch, BE)
    a = _seg_softmax(alpha, dst, N)
    c = _weighted_edges(vs, a, ea, We, ch, BE)
    outm = jax.ops.segment_sum(c, dst, num_segments=N)

    h, gate = _combine(outm, xr, Wb, ln_g, ln_b, gate_W, gate_b, ch, 1000)
    return h, gate


def kernel(x, edge_attr, Wq1, bq1, Wk1, bk1, Wv1, bv1, We1, Ws1, bs1, Wb1,
           Wq2, bq2, Wk2, bk2, Wv2, bv2, We2, Ws2, bs2, Wb2,
           Wq3, bq3, Wk3, bk3, Wv3, bv3, We3, Ws3, bs3, Wb3,
           ln1_g, ln1_b, ln2_g, ln2_b, ln3_g, ln3_b,
           gate_W, gate_b, fc_W, fc_b, edge_index, batch):
    src = edge_index[0]
    dst = edge_index[1]
    zW = jnp.zeros((512, 1), jnp.float32)
    zb = jnp.zeros((1,), jnp.float32)

    h, _ = _layer(x, src, dst, edge_attr, Wq1, bq1, Wk1, bk1, Wv1, bv1,
                  We1, Ws1, bs1, Wb1, ln1_g, ln1_b, zW, zb, 512, 400)
    h, _ = _layer(h, src, dst, edge_attr, Wq2, bq2, Wk2, bk2, Wv2, bv2,
                  We2, Ws2, bs2, Wb2, ln2_g, ln2_b, zW[:256], zb, 256, 800)
    h, gate = _layer(h, src, dst, edge_attr, Wq3, bq3, Wk3, bk3, Wv3, bv3,
                     We3, Ws3, bs3, Wb3, ln3_g, ln3_b, gate_W, gate_b, 64, 1600)

    a = _seg_softmax(gate[:, 0], batch, G)
    pooled = jax.ops.segment_sum(a[:, None] * h, batch, num_segments=G)
    return pooled @ fc_W + fc_b
